# baseline (speedup 1.0000x reference)
import functools

import jax
import jax.numpy as jnp
from jax.experimental import pallas as pl
from jax.experimental.pallas import tpu as pltpu


def _body(uft_ref, ift_ref, uwt_ref, iwt_ref, gb_ref, out_ref, *, d):
    uft = uft_ref[...].astype(jnp.bfloat16)
    ift = ift_ref[...].astype(jnp.bfloat16)
    uwt = uwt_ref[...].astype(jnp.bfloat16)
    iwt = iwt_ref[...].astype(jnp.bfloat16)
    ul = jnp.dot(uwt, uft, preferred_element_type=jnp.float32)
    il = jnp.dot(iwt, ift, preferred_element_type=jnp.float32)
    inter = jnp.sum(ul[:d] * il[:d], axis=0, keepdims=True)
    res = inter + ul[d:d + 1] + il[d:d + 1] + gb_ref[0, 0]
    out_ref[...] = res.reshape(1, 1, res.shape[1])


def kernel(user_features, item_features, user_latent_w, item_latent_w,
           item_biases_w, user_biases_w, global_bias):
    b, nuf = user_features.shape
    nif = item_features.shape[1]
    d = user_latent_w.shape[1]
    bn = 512
    grid = (b // bn,)

    uft = user_features.T
    ift = item_features.T
    uwt = jnp.concatenate([user_latent_w, user_biases_w], axis=1).T
    iwt = jnp.concatenate([item_latent_w, item_biases_w], axis=1).T
    gb2 = global_bias.reshape(1, 1)

    out = pl.pallas_call(
        functools.partial(_body, d=d),
        grid=grid,
        in_specs=[
            pl.BlockSpec((nuf, bn), lambda i: (0, i)),
            pl.BlockSpec((nif, bn), lambda i: (0, i)),
            pl.BlockSpec((d + 1, nuf), lambda i: (0, 0)),
            pl.BlockSpec((d + 1, nif), lambda i: (0, 0)),
            pl.BlockSpec((1, 1), lambda i: (0, 0)),
        ],
        out_specs=pl.BlockSpec((1, 1, bn), lambda i: (i, 0, 0)),
        out_shape=jax.ShapeDtypeStruct((b // bn, 1, bn), jnp.float32),
        compiler_params=pltpu.CompilerParams(
            dimension_semantics=("arbitrary",),
        ),
    )(uft, ift, uwt, iwt, gb2)
    return out.reshape(b)


# speedup vs baseline: 1.1322x; 1.1322x over previous
import functools

import jax
import jax.numpy as jnp
from jax.experimental import pallas as pl
from jax.experimental.pallas import tpu as pltpu


def _body(uft_ref, ift_ref, uwt_ref, iwt_ref, gb_ref, out_ref, *, d):
    uft = uft_ref[...].astype(jnp.bfloat16)
    ift = ift_ref[...].astype(jnp.bfloat16)
    uwt = uwt_ref[...].astype(jnp.bfloat16)
    iwt = iwt_ref[...].astype(jnp.bfloat16)
    ul = jnp.dot(uwt, uft, preferred_element_type=jnp.float32)
    il = jnp.dot(iwt, ift, preferred_element_type=jnp.float32)
    inter = jnp.sum(ul[:d] * il[:d], axis=0, keepdims=True)
    res = inter + ul[d:d + 1] + il[d:d + 1] + gb_ref[0, 0]
    out_ref[...] = res.reshape(1, 1, res.shape[1])


def kernel(user_features, item_features, user_latent_w, item_latent_w,
           item_biases_w, user_biases_w, global_bias):
    b, nuf = user_features.shape
    nif = item_features.shape[1]
    d = user_latent_w.shape[1]
    bn = 1024
    grid = (b // bn,)

    uft = user_features.T
    ift = item_features.T
    uwt = jnp.concatenate([user_latent_w, user_biases_w], axis=1).T
    iwt = jnp.concatenate([item_latent_w, item_biases_w], axis=1).T
    gb2 = global_bias.reshape(1, 1)

    out = pl.pallas_call(
        functools.partial(_body, d=d),
        grid=grid,
        in_specs=[
            pl.BlockSpec((nuf, bn), lambda i: (0, i)),
            pl.BlockSpec((nif, bn), lambda i: (0, i)),
            pl.BlockSpec((d + 1, nuf), lambda i: (0, 0)),
            pl.BlockSpec((d + 1, nif), lambda i: (0, 0)),
            pl.BlockSpec((1, 1), lambda i: (0, 0)),
        ],
        out_specs=pl.BlockSpec((1, 1, bn), lambda i: (i, 0, 0)),
        out_shape=jax.ShapeDtypeStruct((b // bn, 1, bn), jnp.float32),
        compiler_params=pltpu.CompilerParams(
            dimension_semantics=("parallel",),
        ),
    )(uft, ift, uwt, iwt, gb2)
    return out.reshape(b)
